# Initial kernel scaffold; baseline (speedup 1.0000x reference)
import functools

import jax
import jax.numpy as jnp
from jax import lax
from jax.experimental import pallas as pl
from jax.experimental.pallas import tpu as pltpu
from jax.experimental.pallas import tpu_sc as plsc

N = 10000
E = 320000
D = 128
L = 3
EPS = 1e-5

NC = 2
NS = 16
NW = NC * NS
EPW = E // NW
K = 80
NP = 10240
RPT = NP // NS


def _sc_body(m_hbm, src_hbm, dst_hbm, zeros_hbm, out_hbm,
             src_v, dst_v, rows_v, zbuf, acc_sh, sem):
    cid = lax.axis_index("c")
    sid = lax.axis_index("s")
    wid = sid * NC + cid
    rbase = sid * RPT

    pltpu.sync_copy(zeros_hbm, zbuf)
    pltpu.sync_copy(zbuf, acc_sh.at[pl.ds(rbase, RPT)])
    plsc.subcore_barrier()

    ebase = wid * EPW

    def body(c, carry):
        off = ebase + c * K
        pltpu.sync_copy(src_hbm.at[pl.ds(off, K)], src_v)
        pltpu.sync_copy(dst_hbm.at[pl.ds(off, K)], dst_v)
        pltpu.async_copy(m_hbm.at[src_v], rows_v, sem).wait()
        pltpu.sync_copy(rows_v, acc_sh.at[dst_v], add=True)
        return carry

    lax.fori_loop(0, EPW // K, body, 0)
    plsc.subcore_barrier()

    pltpu.sync_copy(acc_sh.at[pl.ds(rbase, RPT)], zbuf)
    pltpu.sync_copy(zbuf, out_hbm.at[pl.ds(cid * NP + rbase, RPT)])


_sc_segsum = functools.partial(
    pl.kernel,
    out_type=jax.ShapeDtypeStruct((NC * NP, D), jnp.float32),
    mesh=plsc.VectorSubcoreMesh(
        core_axis_name="c", subcore_axis_name="s",
        num_cores=NC, num_subcores=NS),
    scratch_types=[
        pltpu.VMEM((K,), jnp.int32),
        pltpu.VMEM((K,), jnp.int32),
        pltpu.VMEM((K, D), jnp.float32),
        pltpu.VMEM((RPT, D), jnp.float32),
        pltpu.VMEM_SHARED((NP, D), jnp.float32),
        pltpu.SemaphoreType.DMA,
    ],
)(_sc_body)


R = 1000


def _pre_body(x_ref, w1_ref, b1_ref, g1_ref, be1_ref, g0_ref, h_ref, m_ref):
    h = lax.dot_general(x_ref[...], w1_ref[...], (((1,), (1,)), ((), ())),
                        preferred_element_type=jnp.float32)
    h = (h + b1_ref[...]) * (g1_ref[...] / jnp.sqrt(1.0 + EPS)) + be1_ref[...]
    h_ref[...] = h
    m_ref[...] = jnp.dot(h, g0_ref[...], preferred_element_type=jnp.float32)


def _tc_pre(x, w1, b1, g1, be1, g0):
    full = lambda i: (0, 0)
    return pl.pallas_call(
        _pre_body,
        grid=(N // R,),
        in_specs=[
            pl.BlockSpec((R, D), lambda i: (i, 0)),
            pl.BlockSpec((D, D), full),
            pl.BlockSpec((1, D), full),
            pl.BlockSpec((1, D), full),
            pl.BlockSpec((1, D), full),
            pl.BlockSpec((D, D), full),
        ],
        out_specs=[pl.BlockSpec((R, D), lambda i: (i, 0)),
                   pl.BlockSpec((R, D), lambda i: (i, 0))],
        out_shape=[jax.ShapeDtypeStruct((N, D), jnp.float32),
                   jax.ShapeDtypeStruct((N, D), jnp.float32)],
    )(x, w1, b1, g1, be1, g0)


def _gru_update(p_ref, h_ref, wih_ref, whh_ref, bih_ref, bhh_ref):
    agg = p_ref[0] + p_ref[1]
    h = h_ref[...]
    gi = lax.dot_general(agg, wih_ref[...], (((1,), (1,)), ((), ())),
                         preferred_element_type=jnp.float32) + bih_ref[...]
    gh = lax.dot_general(h, whh_ref[...], (((1,), (1,)), ((), ())),
                         preferred_element_type=jnp.float32) + bhh_ref[...]
    r = jax.nn.sigmoid(gi[:, :D] + gh[:, :D])
    z = jax.nn.sigmoid(gi[:, D:2 * D] + gh[:, D:2 * D])
    n = jnp.tanh(gi[:, 2 * D:] + r * gh[:, 2 * D:])
    return (1.0 - z) * n + z * h


def _gru_body(p_ref, h_ref, wih_ref, whh_ref, bih_ref, bhh_ref, gn_ref,
              hn_ref, mn_ref):
    hn = _gru_update(p_ref, h_ref, wih_ref, whh_ref, bih_ref, bhh_ref)
    hn_ref[...] = hn
    mn_ref[...] = jnp.dot(hn, gn_ref[...], preferred_element_type=jnp.float32)


def _tc_gru(p, h, wih, whh, bih, bhh, gn):
    full = lambda i: (0, 0)
    return pl.pallas_call(
        _gru_body,
        grid=(N // R,),
        in_specs=[
            pl.BlockSpec((2, R, D), lambda i: (0, i, 0)),
            pl.BlockSpec((R, D), lambda i: (i, 0)),
            pl.BlockSpec((3 * D, D), full),
            pl.BlockSpec((3 * D, D), full),
            pl.BlockSpec((1, 3 * D), full),
            pl.BlockSpec((1, 3 * D), full),
            pl.BlockSpec((D, D), full),
        ],
        out_specs=[pl.BlockSpec((R, D), lambda i: (i, 0)),
                   pl.BlockSpec((R, D), lambda i: (i, 0))],
        out_shape=[jax.ShapeDtypeStruct((N, D), jnp.float32),
                   jax.ShapeDtypeStruct((N, D), jnp.float32)],
    )(p, h, wih, whh, bih, bhh, gn)


def _fin_body(p_ref, h_ref, wih_ref, whh_ref, bih_ref, bhh_ref,
              g2_ref, be2_ref, w2_ref, b2_ref, out_ref, emb_ref):
    hn = _gru_update(p_ref, h_ref, wih_ref, whh_ref, bih_ref, bhh_ref)
    hb = hn * (g2_ref[...] / jnp.sqrt(1.0 + EPS)) + be2_ref[...]
    emb = lax.dot_general(hb, w2_ref[...], (((1,), (1,)), ((), ())),
                          preferred_element_type=jnp.float32) + b2_ref[...]
    emb_ref[...] = emb
    s = emb - jnp.max(emb, axis=1, keepdims=True)
    out_ref[...] = s - jnp.log(jnp.sum(jnp.exp(s), axis=1, keepdims=True))


def _tc_fin(p, h, wih, whh, bih, bhh, g2, be2, w2, b2):
    full = lambda i: (0, 0)
    return pl.pallas_call(
        _fin_body,
        grid=(N // R,),
        in_specs=[
            pl.BlockSpec((2, R, D), lambda i: (0, i, 0)),
            pl.BlockSpec((R, D), lambda i: (i, 0)),
            pl.BlockSpec((3 * D, D), full),
            pl.BlockSpec((3 * D, D), full),
            pl.BlockSpec((1, 3 * D), full),
            pl.BlockSpec((1, 3 * D), full),
            pl.BlockSpec((1, D), full),
            pl.BlockSpec((1, D), full),
            pl.BlockSpec((D, D), full),
            pl.BlockSpec((1, D), full),
        ],
        out_specs=[pl.BlockSpec((R, D), lambda i: (i, 0)),
                   pl.BlockSpec((R, D), lambda i: (i, 0))],
        out_shape=[jax.ShapeDtypeStruct((N, D), jnp.float32),
                   jax.ShapeDtypeStruct((N, D), jnp.float32)],
    )(p, h, wih, whh, bih, bhh, g2, be2, w2, b2)


def kernel(x, edge_index, lin1_W, lin1_b, bn1_gamma, bn1_beta, ggc_weight,
           gru_W_ih, gru_W_hh, gru_b_ih, gru_b_hh, bn2_gamma, bn2_beta,
           lin2_W, lin2_b):
    src = edge_index[0]
    dst = edge_index[1]
    zeros = jnp.zeros((RPT, D), jnp.float32)
    b1 = lin1_b.reshape(1, D)
    g1 = bn1_gamma.reshape(1, D)
    be1 = bn1_beta.reshape(1, D)
    bih = gru_b_ih.reshape(1, 3 * D)
    bhh = gru_b_hh.reshape(1, 3 * D)
    g2 = bn2_gamma.reshape(1, D)
    be2 = bn2_beta.reshape(1, D)
    b2 = lin2_b.reshape(1, D)

    h, m = _tc_pre(x, lin1_W, b1, g1, be1, ggc_weight[0])
    for i in range(L):
        p = _sc_segsum(m, src, dst, zeros).reshape(2, NP, D)
        if i < L - 1:
            h, m = _tc_gru(p, h, gru_W_ih, gru_W_hh, bih, bhh,
                           ggc_weight[i + 1])
        else:
            out, emb = _tc_fin(p, h, gru_W_ih, gru_W_hh, bih, bhh,
                               g2, be2, lin2_W, b2)
    return (out, emb)


# speedup vs baseline: 2.4861x; 2.4861x over previous
import functools

import jax
import jax.numpy as jnp
from jax import lax
from jax.experimental import pallas as pl
from jax.experimental.pallas import tpu as pltpu
from jax.experimental.pallas import tpu_sc as plsc

N = 10000
E = 320000
D = 128
L = 3
EPS = 1e-5

NC = 2
NS = 16
K = 80
NP = 10240
HN = NP // NC
RPT = HN // NS
EPT = E // NS


def _sc_body(m_hbm, src_hbm, dst_hbm, zeros_hbm, out_hbm,
             src_v, dst_v, fsrc_v, fdst_v, rows_v, zbuf, acc_sh, sem):
    cid = lax.axis_index("c")
    sid = lax.axis_index("s")
    rbase = sid * RPT
    lo = cid * HN

    pltpu.sync_copy(zeros_hbm, zbuf)
    pltpu.sync_copy(zbuf, acc_sh.at[pl.ds(rbase, RPT)])
    plsc.subcore_barrier()

    ebase = sid * EPT
    neg1 = jnp.full((16,), -1, jnp.int32)

    def body(c, carry):
        off = ebase + c * K
        pltpu.sync_copy(src_hbm.at[pl.ds(off, K)], src_v)
        pltpu.sync_copy(dst_hbm.at[pl.ds(off, K)], dst_v)
        for j in range(K // 16):
            sl = pl.ds(j * 16, 16)
            dvl = dst_v[sl] - lo
            ok = (dvl >= 0) & (dvl < HN)
            fsrc_v[sl] = jnp.where(ok, src_v[sl], neg1)
            fdst_v[sl] = jnp.where(ok, dvl, neg1)
        pltpu.async_copy(
            m_hbm.at[plsc.Indices(fsrc_v, ignored_value=-1)], rows_v,
            sem).wait()
        pltpu.sync_copy(
            rows_v, acc_sh.at[plsc.Indices(fdst_v, ignored_value=-1)],
            add=True)
        return carry

    lax.fori_loop(0, EPT // K, body, 0)
    plsc.subcore_barrier()

    pltpu.sync_copy(acc_sh.at[pl.ds(rbase, RPT)], zbuf)
    pltpu.sync_copy(zbuf, out_hbm.at[pl.ds(lo + rbase, RPT)])


@functools.cache
def _sc_segsum_kernel():
    return pl.kernel(
        _sc_body,
        out_type=jax.ShapeDtypeStruct((NP, D), jnp.float32),
        mesh=plsc.VectorSubcoreMesh(
            core_axis_name="c", subcore_axis_name="s",
            num_cores=NC, num_subcores=NS),
        scratch_types=[
            pltpu.VMEM((K,), jnp.int32),
            pltpu.VMEM((K,), jnp.int32),
            pltpu.VMEM((K,), jnp.int32),
            pltpu.VMEM((K,), jnp.int32),
            pltpu.VMEM((K, D), jnp.float32),
            pltpu.VMEM((RPT, D), jnp.float32),
            pltpu.VMEM_SHARED((HN, D), jnp.float32),
            pltpu.SemaphoreType.DMA,
        ],
    )


def _sc_segsum(m, src, dst, zeros):
    return _sc_segsum_kernel()(m, src, dst, zeros)


R = 1000


def _pre_body(x_ref, w1_ref, b1_ref, g1_ref, be1_ref, g0_ref, h_ref, m_ref):
    h = lax.dot_general(x_ref[...], w1_ref[...], (((1,), (1,)), ((), ())),
                        preferred_element_type=jnp.float32)
    h = (h + b1_ref[...]) * (g1_ref[...] / jnp.sqrt(1.0 + EPS)) + be1_ref[...]
    h_ref[...] = h
    m_ref[...] = jnp.dot(h, g0_ref[...], preferred_element_type=jnp.float32)


def _tc_pre(x, w1, b1, g1, be1, g0):
    full = lambda i: (0, 0)
    return pl.pallas_call(
        _pre_body,
        grid=(N // R,),
        in_specs=[
            pl.BlockSpec((R, D), lambda i: (i, 0)),
            pl.BlockSpec((D, D), full),
            pl.BlockSpec((1, D), full),
            pl.BlockSpec((1, D), full),
            pl.BlockSpec((1, D), full),
            pl.BlockSpec((D, D), full),
        ],
        out_specs=[pl.BlockSpec((R, D), lambda i: (i, 0)),
                   pl.BlockSpec((R, D), lambda i: (i, 0))],
        out_shape=[jax.ShapeDtypeStruct((N, D), jnp.float32),
                   jax.ShapeDtypeStruct((N, D), jnp.float32)],
    )(x, w1, b1, g1, be1, g0)


def _gru_update(p_ref, h_ref, wih_ref, whh_ref, bih_ref, bhh_ref):
    agg = p_ref[...]
    h = h_ref[...]
    gi = lax.dot_general(agg, wih_ref[...], (((1,), (1,)), ((), ())),
                         preferred_element_type=jnp.float32) + bih_ref[...]
    gh = lax.dot_general(h, whh_ref[...], (((1,), (1,)), ((), ())),
                         preferred_element_type=jnp.float32) + bhh_ref[...]
    r = jax.nn.sigmoid(gi[:, :D] + gh[:, :D])
    z = jax.nn.sigmoid(gi[:, D:2 * D] + gh[:, D:2 * D])
    n = jnp.tanh(gi[:, 2 * D:] + r * gh[:, 2 * D:])
    return (1.0 - z) * n + z * h


def _gru_body(p_ref, h_ref, wih_ref, whh_ref, bih_ref, bhh_ref, gn_ref,
              hn_ref, mn_ref):
    hn = _gru_update(p_ref, h_ref, wih_ref, whh_ref, bih_ref, bhh_ref)
    hn_ref[...] = hn
    mn_ref[...] = jnp.dot(hn, gn_ref[...], preferred_element_type=jnp.float32)


def _tc_gru(p, h, wih, whh, bih, bhh, gn):
    full = lambda i: (0, 0)
    return pl.pallas_call(
        _gru_body,
        grid=(N // R,),
        in_specs=[
            pl.BlockSpec((R, D), lambda i: (i, 0)),
            pl.BlockSpec((R, D), lambda i: (i, 0)),
            pl.BlockSpec((3 * D, D), full),
            pl.BlockSpec((3 * D, D), full),
            pl.BlockSpec((1, 3 * D), full),
            pl.BlockSpec((1, 3 * D), full),
            pl.BlockSpec((D, D), full),
        ],
        out_specs=[pl.BlockSpec((R, D), lambda i: (i, 0)),
                   pl.BlockSpec((R, D), lambda i: (i, 0))],
        out_shape=[jax.ShapeDtypeStruct((N, D), jnp.float32),
                   jax.ShapeDtypeStruct((N, D), jnp.float32)],
    )(p, h, wih, whh, bih, bhh, gn)


def _fin_body(h_ref, g2_ref, be2_ref, w2_ref, b2_ref, out_ref, emb_ref):
    hb = h_ref[...] * (g2_ref[...] / jnp.sqrt(1.0 + EPS)) + be2_ref[...]
    emb = lax.dot_general(hb, w2_ref[...], (((1,), (1,)), ((), ())),
                          preferred_element_type=jnp.float32) + b2_ref[...]
    emb_ref[...] = emb
    s = emb - jnp.max(emb, axis=1, keepdims=True)
    out_ref[...] = s - jnp.log(jnp.sum(jnp.exp(s), axis=1, keepdims=True))


def _tc_fin(h, g2, be2, w2, b2):
    full = lambda i: (0, 0)
    return pl.pallas_call(
        _fin_body,
        grid=(N // R,),
        in_specs=[
            pl.BlockSpec((R, D), lambda i: (i, 0)),
            pl.BlockSpec((1, D), full),
            pl.BlockSpec((1, D), full),
            pl.BlockSpec((D, D), full),
            pl.BlockSpec((1, D), full),
        ],
        out_specs=[pl.BlockSpec((R, D), lambda i: (i, 0)),
                   pl.BlockSpec((R, D), lambda i: (i, 0))],
        out_shape=[jax.ShapeDtypeStruct((N, D), jnp.float32),
                   jax.ShapeDtypeStruct((N, D), jnp.float32)],
    )(h, g2, be2, w2, b2)


def kernel(x, edge_index, lin1_W, lin1_b, bn1_gamma, bn1_beta, ggc_weight,
           gru_W_ih, gru_W_hh, gru_b_ih, gru_b_hh, bn2_gamma, bn2_beta,
           lin2_W, lin2_b):
    src = edge_index[0]
    dst = edge_index[1]
    zeros = jnp.zeros((RPT, D), jnp.float32)
    b1 = lin1_b.reshape(1, D)
    g1 = bn1_gamma.reshape(1, D)
    be1 = bn1_beta.reshape(1, D)
    bih = gru_b_ih.reshape(1, 3 * D)
    bhh = gru_b_hh.reshape(1, 3 * D)
    g2 = bn2_gamma.reshape(1, D)
    be2 = bn2_beta.reshape(1, D)
    b2 = lin2_b.reshape(1, D)

    h, m = _tc_pre(x, lin1_W, b1, g1, be1, ggc_weight[0])

    gn_stack = jnp.concatenate([ggc_weight[1:], ggc_weight[:1]], axis=0)

    def step(carry, gn):
        h, m = carry
        p = _sc_segsum(m, src, dst, zeros)
        h, m = _tc_gru(p, h, gru_W_ih, gru_W_hh, bih, bhh, gn)
        return (h, m), None

    (h, _), _ = lax.scan(step, (h, m), gn_stack)
    out, emb = _tc_fin(h, g2, be2, lin2_W, b2)
    return (out, emb)


# speedup vs baseline: 4.9996x; 2.0110x over previous
import functools

import jax
import jax.numpy as jnp
from jax import lax
from jax.experimental import pallas as pl
from jax.experimental.pallas import tpu as pltpu
from jax.experimental.pallas import tpu_sc as plsc

N = 10000
E = 320000
D = 128
L = 3
EPS = 1e-5

NC = 2
NS = 16
K = 80
NP = 10240
HN = NP // NC
RPT = HN // NS
EPT = E // NS
NCH = EPT // K


def _sc_body(m_hbm, src_hbm, dst_hbm, out_hbm,
             rsrc_v, rdst_v, fsrc_v, fdst_v, rows_v, acc_sh,
             gsem0, gsem1, ssem0, ssem1):
    gsem = (gsem0, gsem1)
    ssem = (ssem0, ssem1)
    cid = lax.axis_index("c")
    sid = lax.axis_index("s")
    rbase = sid * RPT
    lo = cid * HN
    ebase = sid * EPT
    neg1 = jnp.full((16,), -1, jnp.int32)
    zeros16 = jnp.zeros((16,), jnp.float32)

    def zrow(r, carry):
        for j in range(D // 16):
            rows_v[0, r, pl.ds(j * 16, 16)] = zeros16
        return carry
    lax.fori_loop(0, K, zrow, 0)
    for q in range(RPT // K):
        pltpu.sync_copy(rows_v.at[0],
                        acc_sh.at[pl.ds(rbase + q * K, K)])

    pltpu.sync_copy(src_hbm.at[pl.ds(ebase, EPT)], rsrc_v)
    pltpu.sync_copy(dst_hbm.at[pl.ds(ebase, EPT)], rdst_v)

    def filter_chunk(c, b):
        for j in range(K // 16):
            sl = pl.ds(c * K + j * 16, 16)
            dvl = rdst_v[sl] - lo
            ok = (dvl >= 0) & (dvl < HN)
            fsrc_v[b, pl.ds(j * 16, 16)] = jnp.where(ok, rsrc_v[sl], neg1)
            fdst_v[b, pl.ds(j * 16, 16)] = jnp.where(ok, dvl, neg1)

    def fire_gather(b):
        pltpu.async_copy(
            m_hbm.at[plsc.Indices(fsrc_v.at[b], ignored_value=-1)],
            rows_v.at[b], gsem[b])

    def drain_gather(b):
        pltpu.make_async_copy(
            m_hbm.at[plsc.Indices(fsrc_v.at[b], ignored_value=-1)],
            rows_v.at[b], gsem[b]).wait()

    def fire_scatter(b):
        pltpu.async_copy(
            rows_v.at[b],
            acc_sh.at[plsc.Indices(fdst_v.at[b], ignored_value=-1)],
            ssem[b], add=True)

    def drain_scatter(b):
        pltpu.make_async_copy(
            rows_v.at[b],
            acc_sh.at[plsc.Indices(fdst_v.at[b], ignored_value=-1)],
            ssem[b]).wait()

    filter_chunk(0, 0)
    fire_gather(0)
    plsc.subcore_barrier()

    def body(c2, carry):
        for b in (0, 1):
            c = c2 * 2 + b
            drain_gather(b)

            @pl.when(c > 0)
            def _():
                drain_scatter(1 - b)
            fire_scatter(b)

            @pl.when(c + 1 < NCH)
            def _():
                filter_chunk(c + 1, 1 - b)
                fire_gather(1 - b)
        return carry

    lax.fori_loop(0, NCH // 2, body, 0)
    drain_scatter(1)
    plsc.subcore_barrier()

    for q in range(RPT // K):
        pltpu.sync_copy(acc_sh.at[pl.ds(rbase + q * K, K)], rows_v.at[0])
        pltpu.sync_copy(rows_v.at[0],
                        out_hbm.at[pl.ds(lo + rbase + q * K, K)])


@functools.cache
def _sc_segsum_kernel():
    return pl.kernel(
        _sc_body,
        out_type=jax.ShapeDtypeStruct((NP, D), jnp.float32),
        mesh=plsc.VectorSubcoreMesh(
            core_axis_name="c", subcore_axis_name="s",
            num_cores=NC, num_subcores=NS),
        scratch_types=[
            pltpu.VMEM((EPT,), jnp.int32),
            pltpu.VMEM((EPT,), jnp.int32),
            pltpu.VMEM((2, K), jnp.int32),
            pltpu.VMEM((2, K), jnp.int32),
            pltpu.VMEM((2, K, D), jnp.float32),
            pltpu.VMEM_SHARED((HN, D), jnp.float32),
            pltpu.SemaphoreType.DMA,
            pltpu.SemaphoreType.DMA,
            pltpu.SemaphoreType.DMA,
            pltpu.SemaphoreType.DMA,
        ],
    )


def _sc_segsum(m, src, dst):
    return _sc_segsum_kernel()(m, src, dst)


R = 1000


def _pre_body(x_ref, w1_ref, b1_ref, g1_ref, be1_ref, g0_ref, h_ref, m_ref):
    h = lax.dot_general(x_ref[...], w1_ref[...], (((1,), (1,)), ((), ())),
                        preferred_element_type=jnp.float32)
    h = (h + b1_ref[...]) * (g1_ref[...] / jnp.sqrt(1.0 + EPS)) + be1_ref[...]
    h_ref[...] = h
    m_ref[...] = jnp.dot(h, g0_ref[...], preferred_element_type=jnp.float32)


def _tc_pre(x, w1, b1, g1, be1, g0):
    full = lambda i: (0, 0)
    return pl.pallas_call(
        _pre_body,
        grid=(N // R,),
        in_specs=[
            pl.BlockSpec((R, D), lambda i: (i, 0)),
            pl.BlockSpec((D, D), full),
            pl.BlockSpec((1, D), full),
            pl.BlockSpec((1, D), full),
            pl.BlockSpec((1, D), full),
            pl.BlockSpec((D, D), full),
        ],
        out_specs=[pl.BlockSpec((R, D), lambda i: (i, 0)),
                   pl.BlockSpec((R, D), lambda i: (i, 0))],
        out_shape=[jax.ShapeDtypeStruct((N, D), jnp.float32),
                   jax.ShapeDtypeStruct((N, D), jnp.float32)],
    )(x, w1, b1, g1, be1, g0)


def _gru_update(p_ref, h_ref, wih_ref, whh_ref, bih_ref, bhh_ref):
    agg = p_ref[...]
    h = h_ref[...]
    gi = lax.dot_general(agg, wih_ref[...], (((1,), (1,)), ((), ())),
                         preferred_element_type=jnp.float32) + bih_ref[...]
    gh = lax.dot_general(h, whh_ref[...], (((1,), (1,)), ((), ())),
                         preferred_element_type=jnp.float32) + bhh_ref[...]
    r = jax.nn.sigmoid(gi[:, :D] + gh[:, :D])
    z = jax.nn.sigmoid(gi[:, D:2 * D] + gh[:, D:2 * D])
    n = jnp.tanh(gi[:, 2 * D:] + r * gh[:, 2 * D:])
    return (1.0 - z) * n + z * h


def _gru_body(p_ref, h_ref, wih_ref, whh_ref, bih_ref, bhh_ref, gn_ref,
              hn_ref, mn_ref):
    hn = _gru_update(p_ref, h_ref, wih_ref, whh_ref, bih_ref, bhh_ref)
    hn_ref[...] = hn
    mn_ref[...] = jnp.dot(hn, gn_ref[...], preferred_element_type=jnp.float32)


def _tc_gru(p, h, wih, whh, bih, bhh, gn):
    full = lambda i: (0, 0)
    return pl.pallas_call(
        _gru_body,
        grid=(N // R,),
        in_specs=[
            pl.BlockSpec((R, D), lambda i: (i, 0)),
            pl.BlockSpec((R, D), lambda i: (i, 0)),
            pl.BlockSpec((3 * D, D), full),
            pl.BlockSpec((3 * D, D), full),
            pl.BlockSpec((1, 3 * D), full),
            pl.BlockSpec((1, 3 * D), full),
            pl.BlockSpec((D, D), full),
        ],
        out_specs=[pl.BlockSpec((R, D), lambda i: (i, 0)),
                   pl.BlockSpec((R, D), lambda i: (i, 0))],
        out_shape=[jax.ShapeDtypeStruct((N, D), jnp.float32),
                   jax.ShapeDtypeStruct((N, D), jnp.float32)],
    )(p, h, wih, whh, bih, bhh, gn)


def _fin_body(h_ref, g2_ref, be2_ref, w2_ref, b2_ref, out_ref, emb_ref):
    hb = h_ref[...] * (g2_ref[...] / jnp.sqrt(1.0 + EPS)) + be2_ref[...]
    emb = lax.dot_general(hb, w2_ref[...], (((1,), (1,)), ((), ())),
                          preferred_element_type=jnp.float32) + b2_ref[...]
    emb_ref[...] = emb
    s = emb - jnp.max(emb, axis=1, keepdims=True)
    out_ref[...] = s - jnp.log(jnp.sum(jnp.exp(s), axis=1, keepdims=True))


def _tc_fin(h, g2, be2, w2, b2):
    full = lambda i: (0, 0)
    return pl.pallas_call(
        _fin_body,
        grid=(N // R,),
        in_specs=[
            pl.BlockSpec((R, D), lambda i: (i, 0)),
            pl.BlockSpec((1, D), full),
            pl.BlockSpec((1, D), full),
            pl.BlockSpec((D, D), full),
            pl.BlockSpec((1, D), full),
        ],
        out_specs=[pl.BlockSpec((R, D), lambda i: (i, 0)),
                   pl.BlockSpec((R, D), lambda i: (i, 0))],
        out_shape=[jax.ShapeDtypeStruct((N, D), jnp.float32),
                   jax.ShapeDtypeStruct((N, D), jnp.float32)],
    )(h, g2, be2, w2, b2)


def kernel(x, edge_index, lin1_W, lin1_b, bn1_gamma, bn1_beta, ggc_weight,
           gru_W_ih, gru_W_hh, gru_b_ih, gru_b_hh, bn2_gamma, bn2_beta,
           lin2_W, lin2_b):
    src = edge_index[0]
    dst = edge_index[1]
    b1 = lin1_b.reshape(1, D)
    g1 = bn1_gamma.reshape(1, D)
    be1 = bn1_beta.reshape(1, D)
    bih = gru_b_ih.reshape(1, 3 * D)
    bhh = gru_b_hh.reshape(1, 3 * D)
    g2 = bn2_gamma.reshape(1, D)
    be2 = bn2_beta.reshape(1, D)
    b2 = lin2_b.reshape(1, D)

    h, m = _tc_pre(x, lin1_W, b1, g1, be1, ggc_weight[0])

    gn_stack = jnp.concatenate([ggc_weight[1:], ggc_weight[:1]], axis=0)

    def step(carry, gn):
        h, m = carry
        p = _sc_segsum(m, src, dst)
        h, m = _tc_gru(p, h, gru_W_ih, gru_W_hh, bih, bhh, gn)
        return (h, m), None

    (h, _), _ = lax.scan(step, (h, m), gn_stack)
    out, emb = _tc_fin(h, g2, be2, lin2_W, b2)
    return (out, emb)


# speedup vs baseline: 6.2937x; 1.2589x over previous
import functools

import jax
import jax.numpy as jnp
from jax import lax
from jax.experimental import pallas as pl
from jax.experimental.pallas import tpu as pltpu
from jax.experimental.pallas import tpu_sc as plsc

N = 10000
E = 320000
D = 128
L = 3
EPS = 1e-5

NC = 2
NS = 16
K = 128
NP = 10240
HN = NP // NC
RPT = HN // NS
ZB = 64
EPT = E // NS
NCHF = EPT // K
KR = EPT - NCHF * K


def _sc_body(m_hbm, src_hbm, dst_hbm, out_hbm,
             rsrc_v, rdst_v, fsrc_v, fdst_v, frem_v, rows_v, rowr_v, acc_sh,
             gsem0, gsem1, ssem0, ssem1):
    gsem = (gsem0, gsem1)
    ssem = (ssem0, ssem1)
    cid = lax.axis_index("c")
    sid = lax.axis_index("s")
    rbase = sid * RPT
    lo = cid * HN
    ebase = sid * EPT
    neg1 = jnp.full((16,), -1, jnp.int32)
    zeros16 = jnp.zeros((16,), jnp.float32)

    def zrow(r, carry):
        for j in range(D // 16):
            rows_v[0, r, pl.ds(j * 16, 16)] = zeros16
        return carry
    lax.fori_loop(0, ZB, zrow, 0)
    for q in range(RPT // ZB):
        pltpu.sync_copy(rows_v.at[0, pl.ds(0, ZB)],
                        acc_sh.at[pl.ds(rbase + q * ZB, ZB)])

    pltpu.sync_copy(src_hbm.at[pl.ds(ebase, EPT)], rsrc_v)
    pltpu.sync_copy(dst_hbm.at[pl.ds(ebase, EPT)], rdst_v)

    def filter_chunk(c, t):
        for j in range(K // 16):
            sl = pl.ds(c * K + j * 16, 16)
            dvl = rdst_v[sl] - lo
            ok = (dvl >= 0) & (dvl < HN)
            fsrc_v[t, pl.ds(j * 16, 16)] = jnp.where(ok, rsrc_v[sl], neg1)
            fdst_v[t, pl.ds(j * 16, 16)] = jnp.where(ok, dvl, neg1)

    def fire_gather(b, t):
        pltpu.async_copy(
            m_hbm.at[plsc.Indices(fsrc_v.at[t], ignored_value=-1)],
            rows_v.at[b], gsem[b])

    def drain_gather(b, t):
        pltpu.make_async_copy(
            m_hbm.at[plsc.Indices(fsrc_v.at[t], ignored_value=-1)],
            rows_v.at[b], gsem[b]).wait()

    def fire_scatter(b, t):
        pltpu.async_copy(
            rows_v.at[b],
            acc_sh.at[plsc.Indices(fdst_v.at[t], ignored_value=-1)],
            ssem[b], add=True)

    def drain_scatter(b, t):
        pltpu.make_async_copy(
            rows_v.at[b],
            acc_sh.at[plsc.Indices(fdst_v.at[t], ignored_value=-1)],
            ssem[b]).wait()

    filter_chunk(0, 0)
    fire_gather(0, 0)
    plsc.subcore_barrier()

    def body(c6, carry):
        for u in range(6):
            b, t = u % 2, u % 3
            c = c6 * 6 + u

            @pl.when(c + 1 < NCHF)
            def _():
                filter_chunk(c + 1, (t + 1) % 3)
            drain_gather(b, t)

            @pl.when(c > 0)
            def _():
                drain_scatter(1 - b, (t + 2) % 3)
            fire_scatter(b, t)

            @pl.when(c + 1 < NCHF)
            def _():
                fire_gather(1 - b, (t + 1) % 3)
        return carry

    lax.fori_loop(0, NCHF // 6, body, 0)
    drain_scatter(1, 2)

    for j in range(KR // 16):
        sl = pl.ds(NCHF * K + j * 16, 16)
        dvl = rdst_v[sl] - lo
        ok = (dvl >= 0) & (dvl < HN)
        frem_v[0, pl.ds(j * 16, 16)] = jnp.where(ok, rsrc_v[sl], neg1)
        frem_v[1, pl.ds(j * 16, 16)] = jnp.where(ok, dvl, neg1)
    pltpu.async_copy(
        m_hbm.at[plsc.Indices(frem_v.at[0], ignored_value=-1)],
        rowr_v, gsem[0]).wait()
    pltpu.async_copy(
        rowr_v, acc_sh.at[plsc.Indices(frem_v.at[1], ignored_value=-1)],
        ssem[0], add=True).wait()
    plsc.subcore_barrier()

    for q in range(RPT // ZB):
        pltpu.sync_copy(acc_sh.at[pl.ds(rbase + q * ZB, ZB)],
                        rows_v.at[0, pl.ds(0, ZB)])
        pltpu.sync_copy(rows_v.at[0, pl.ds(0, ZB)],
                        out_hbm.at[pl.ds(lo + rbase + q * ZB, ZB)])


@functools.cache
def _sc_segsum_kernel():
    return pl.kernel(
        _sc_body,
        out_type=jax.ShapeDtypeStruct((NP, D), jnp.float32),
        mesh=plsc.VectorSubcoreMesh(
            core_axis_name="c", subcore_axis_name="s",
            num_cores=NC, num_subcores=NS),
        scratch_types=[
            pltpu.VMEM((EPT,), jnp.int32),
            pltpu.VMEM((EPT,), jnp.int32),
            pltpu.VMEM((3, K), jnp.int32),
            pltpu.VMEM((3, K), jnp.int32),
            pltpu.VMEM((2, KR), jnp.int32),
            pltpu.VMEM((2, K, D), jnp.float32),
            pltpu.VMEM((KR, D), jnp.float32),
            pltpu.VMEM_SHARED((HN, D), jnp.float32),
            pltpu.SemaphoreType.DMA,
            pltpu.SemaphoreType.DMA,
            pltpu.SemaphoreType.DMA,
            pltpu.SemaphoreType.DMA,
        ],
    )


def _sc_segsum(m, src, dst):
    return _sc_segsum_kernel()(m, src, dst)


R = 1000


def _pre_body(x_ref, w1_ref, b1_ref, g1_ref, be1_ref, g0_ref, h_ref, m_ref):
    h = lax.dot_general(x_ref[...], w1_ref[...], (((1,), (1,)), ((), ())),
                        preferred_element_type=jnp.float32)
    h = (h + b1_ref[...]) * (g1_ref[...] / jnp.sqrt(1.0 + EPS)) + be1_ref[...]
    h_ref[...] = h
    m_ref[...] = jnp.dot(h, g0_ref[...], preferred_element_type=jnp.float32)


def _tc_pre(x, w1, b1, g1, be1, g0):
    full = lambda i: (0, 0)
    return pl.pallas_call(
        _pre_body,
        grid=(N // R,),
        in_specs=[
            pl.BlockSpec((R, D), lambda i: (i, 0)),
            pl.BlockSpec((D, D), full),
            pl.BlockSpec((1, D), full),
            pl.BlockSpec((1, D), full),
            pl.BlockSpec((1, D), full),
            pl.BlockSpec((D, D), full),
        ],
        out_specs=[pl.BlockSpec((R, D), lambda i: (i, 0)),
                   pl.BlockSpec((R, D), lambda i: (i, 0))],
        out_shape=[jax.ShapeDtypeStruct((N, D), jnp.float32),
                   jax.ShapeDtypeStruct((N, D), jnp.float32)],
    )(x, w1, b1, g1, be1, g0)


def _gru_update(p_ref, h_ref, wih_ref, whh_ref, bih_ref, bhh_ref):
    agg = p_ref[...]
    h = h_ref[...]
    gi = lax.dot_general(agg, wih_ref[...], (((1,), (1,)), ((), ())),
                         preferred_element_type=jnp.float32) + bih_ref[...]
    gh = lax.dot_general(h, whh_ref[...], (((1,), (1,)), ((), ())),
                         preferred_element_type=jnp.float32) + bhh_ref[...]
    r = jax.nn.sigmoid(gi[:, :D] + gh[:, :D])
    z = jax.nn.sigmoid(gi[:, D:2 * D] + gh[:, D:2 * D])
    n = jnp.tanh(gi[:, 2 * D:] + r * gh[:, 2 * D:])
    return (1.0 - z) * n + z * h


def _gru_body(p_ref, h_ref, wih_ref, whh_ref, bih_ref, bhh_ref, gn_ref,
              hn_ref, mn_ref):
    hn = _gru_update(p_ref, h_ref, wih_ref, whh_ref, bih_ref, bhh_ref)
    hn_ref[...] = hn
    mn_ref[...] = jnp.dot(hn, gn_ref[...], preferred_element_type=jnp.float32)


def _tc_gru(p, h, wih, whh, bih, bhh, gn):
    full = lambda i: (0, 0)
    return pl.pallas_call(
        _gru_body,
        grid=(N // R,),
        in_specs=[
            pl.BlockSpec((R, D), lambda i: (i, 0)),
            pl.BlockSpec((R, D), lambda i: (i, 0)),
            pl.BlockSpec((3 * D, D), full),
            pl.BlockSpec((3 * D, D), full),
            pl.BlockSpec((1, 3 * D), full),
            pl.BlockSpec((1, 3 * D), full),
            pl.BlockSpec((D, D), full),
        ],
        out_specs=[pl.BlockSpec((R, D), lambda i: (i, 0)),
                   pl.BlockSpec((R, D), lambda i: (i, 0))],
        out_shape=[jax.ShapeDtypeStruct((N, D), jnp.float32),
                   jax.ShapeDtypeStruct((N, D), jnp.float32)],
    )(p, h, wih, whh, bih, bhh, gn)


def _fin_body(h_ref, g2_ref, be2_ref, w2_ref, b2_ref, out_ref, emb_ref):
    hb = h_ref[...] * (g2_ref[...] / jnp.sqrt(1.0 + EPS)) + be2_ref[...]
    emb = lax.dot_general(hb, w2_ref[...], (((1,), (1,)), ((), ())),
                          preferred_element_type=jnp.float32) + b2_ref[...]
    emb_ref[...] = emb
    s = emb - jnp.max(emb, axis=1, keepdims=True)
    out_ref[...] = s - jnp.log(jnp.sum(jnp.exp(s), axis=1, keepdims=True))


def _tc_fin(h, g2, be2, w2, b2):
    full = lambda i: (0, 0)
    return pl.pallas_call(
        _fin_body,
        grid=(N // R,),
        in_specs=[
            pl.BlockSpec((R, D), lambda i: (i, 0)),
            pl.BlockSpec((1, D), full),
            pl.BlockSpec((1, D), full),
            pl.BlockSpec((D, D), full),
            pl.BlockSpec((1, D), full),
        ],
        out_specs=[pl.BlockSpec((R, D), lambda i: (i, 0)),
                   pl.BlockSpec((R, D), lambda i: (i, 0))],
        out_shape=[jax.ShapeDtypeStruct((N, D), jnp.float32),
                   jax.ShapeDtypeStruct((N, D), jnp.float32)],
    )(h, g2, be2, w2, b2)


def kernel(x, edge_index, lin1_W, lin1_b, bn1_gamma, bn1_beta, ggc_weight,
           gru_W_ih, gru_W_hh, gru_b_ih, gru_b_hh, bn2_gamma, bn2_beta,
           lin2_W, lin2_b):
    src = edge_index[0]
    dst = edge_index[1]
    b1 = lin1_b.reshape(1, D)
    g1 = bn1_gamma.reshape(1, D)
    be1 = bn1_beta.reshape(1, D)
    bih = gru_b_ih.reshape(1, 3 * D)
    bhh = gru_b_hh.reshape(1, 3 * D)
    g2 = bn2_gamma.reshape(1, D)
    be2 = bn2_beta.reshape(1, D)
    b2 = lin2_b.reshape(1, D)

    h, m = _tc_pre(x, lin1_W, b1, g1, be1, ggc_weight[0])

    gn_stack = jnp.concatenate([ggc_weight[1:], ggc_weight[:1]], axis=0)

    def step(carry, gn):
        h, m = carry
        p = _sc_segsum(m, src, dst)
        h, m = _tc_gru(p, h, gru_W_ih, gru_W_hh, bih, bhh, gn)
        return (h, m), None

    (h, _), _ = lax.scan(step, (h, m), gn_stack)
    out, emb = _tc_fin(h, g2, be2, lin2_W, b2)
    return (out, emb)


# speedup vs baseline: 7.6072x; 1.2087x over previous
import functools

import jax
import jax.numpy as jnp
from jax import lax
from jax.experimental import pallas as pl
from jax.experimental.pallas import tpu as pltpu
from jax.experimental.pallas import tpu_sc as plsc

N = 10000
E = 320000
D = 128
L = 3
EPS = 1e-5

NC = 2
NS = 16
K = 64
NR = 4
NP = 10240
HN = NP // NC
RPT = HN // NS
ZB = 64
EPT = E // NS
NCHF = EPT // K
KR = EPT - NCHF * K


def _sc_body(m_hbm, src_hbm, dst_hbm, out_hbm,
             rsrc_v, rdst_v, fsrc_v, fdst_v, frem_v, rows_v, rowr_v, acc_sh,
             gsem0, gsem1, gsem2, gsem3, ssem0, ssem1, ssem2, ssem3):
    gsem = (gsem0, gsem1, gsem2, gsem3)
    ssem = (ssem0, ssem1, ssem2, ssem3)
    cid = lax.axis_index("c")
    sid = lax.axis_index("s")
    rbase = sid * RPT
    lo = cid * HN
    ebase = sid * EPT
    neg1 = jnp.full((16,), -1, jnp.int32)
    zeros16 = jnp.zeros((16,), jnp.float32)

    def zrow(r, carry):
        for j in range(D // 16):
            rows_v[0, r, pl.ds(j * 16, 16)] = zeros16
        return carry
    lax.fori_loop(0, ZB, zrow, 0)
    for q in range(RPT // ZB):
        pltpu.sync_copy(rows_v.at[0, pl.ds(0, ZB)],
                        acc_sh.at[pl.ds(rbase + q * ZB, ZB)])

    pltpu.sync_copy(src_hbm.at[pl.ds(ebase, EPT)], rsrc_v)
    pltpu.sync_copy(dst_hbm.at[pl.ds(ebase, EPT)], rdst_v)

    def filter_chunk(c, t):
        for j in range(K // 16):
            sl = pl.ds(c * K + j * 16, 16)
            dvl = rdst_v[sl] - lo
            ok = (dvl >= 0) & (dvl < HN)
            fsrc_v[t, pl.ds(j * 16, 16)] = jnp.where(ok, rsrc_v[sl], neg1)
            fdst_v[t, pl.ds(j * 16, 16)] = jnp.where(ok, dvl, neg1)

    def fire_gather(t):
        pltpu.async_copy(
            m_hbm.at[plsc.Indices(fsrc_v.at[t], ignored_value=-1)],
            rows_v.at[t], gsem[t])

    def drain_gather(t):
        pltpu.make_async_copy(
            m_hbm.at[plsc.Indices(fsrc_v.at[t], ignored_value=-1)],
            rows_v.at[t], gsem[t]).wait()

    def fire_scatter(t):
        pltpu.async_copy(
            rows_v.at[t],
            acc_sh.at[plsc.Indices(fdst_v.at[t], ignored_value=-1)],
            ssem[t], add=True)

    def drain_scatter(t):
        pltpu.make_async_copy(
            rows_v.at[t],
            acc_sh.at[plsc.Indices(fdst_v.at[t], ignored_value=-1)],
            ssem[t]).wait()

    filter_chunk(0, 0)
    fire_gather(0)
    filter_chunk(1, 1)
    fire_gather(1)
    plsc.subcore_barrier()

    def body(c4, carry):
        for u in range(NR):
            c = c4 * NR + u
            t = u
            tp2 = (u + 2) % NR

            @pl.when(c >= 2)
            def _():
                drain_scatter(tp2)

            @pl.when(c + 2 < NCHF)
            def _():
                filter_chunk(c + 2, tp2)
            drain_gather(t)

            fire_scatter(t)

            @pl.when(c + 2 < NCHF)
            def _():
                fire_gather(tp2)
        return carry

    lax.fori_loop(0, NCHF // NR, body, 0)
    drain_scatter(2)
    drain_scatter(3)

    for j in range(KR // 16):
        sl = pl.ds(NCHF * K + j * 16, 16)
        dvl = rdst_v[sl] - lo
        ok = (dvl >= 0) & (dvl < HN)
        frem_v[0, pl.ds(j * 16, 16)] = jnp.where(ok, rsrc_v[sl], neg1)
        frem_v[1, pl.ds(j * 16, 16)] = jnp.where(ok, dvl, neg1)
    pltpu.async_copy(
        m_hbm.at[plsc.Indices(frem_v.at[0], ignored_value=-1)],
        rowr_v, gsem[0]).wait()
    pltpu.async_copy(
        rowr_v, acc_sh.at[plsc.Indices(frem_v.at[1], ignored_value=-1)],
        ssem[0], add=True).wait()
    plsc.subcore_barrier()

    for q in range(RPT // ZB):
        pltpu.sync_copy(acc_sh.at[pl.ds(rbase + q * ZB, ZB)],
                        rows_v.at[0, pl.ds(0, ZB)])
        pltpu.sync_copy(rows_v.at[0, pl.ds(0, ZB)],
                        out_hbm.at[pl.ds(lo + rbase + q * ZB, ZB)])


@functools.cache
def _sc_segsum_kernel():
    return pl.kernel(
        _sc_body,
        out_type=jax.ShapeDtypeStruct((NP, D), jnp.float32),
        mesh=plsc.VectorSubcoreMesh(
            core_axis_name="c", subcore_axis_name="s",
            num_cores=NC, num_subcores=NS),
        scratch_types=[
            pltpu.VMEM((EPT,), jnp.int32),
            pltpu.VMEM((EPT,), jnp.int32),
            pltpu.VMEM((NR, K), jnp.int32),
            pltpu.VMEM((NR, K), jnp.int32),
            pltpu.VMEM((2, KR), jnp.int32),
            pltpu.VMEM((NR, K, D), jnp.float32),
            pltpu.VMEM((KR, D), jnp.float32),
            pltpu.VMEM_SHARED((HN, D), jnp.float32),
            pltpu.SemaphoreType.DMA,
            pltpu.SemaphoreType.DMA,
            pltpu.SemaphoreType.DMA,
            pltpu.SemaphoreType.DMA,
            pltpu.SemaphoreType.DMA,
            pltpu.SemaphoreType.DMA,
            pltpu.SemaphoreType.DMA,
            pltpu.SemaphoreType.DMA,
        ],
    )


def _sc_segsum(m, src, dst):
    return _sc_segsum_kernel()(m, src, dst)


R = 1000


def _pre_body(x_ref, w1_ref, b1_ref, g1_ref, be1_ref, g0_ref, h_ref, m_ref):
    h = lax.dot_general(x_ref[...], w1_ref[...], (((1,), (1,)), ((), ())),
                        preferred_element_type=jnp.float32)
    h = (h + b1_ref[...]) * (g1_ref[...] / jnp.sqrt(1.0 + EPS)) + be1_ref[...]
    h_ref[...] = h
    m_ref[...] = jnp.dot(h, g0_ref[...], preferred_element_type=jnp.float32)


def _tc_pre(x, w1, b1, g1, be1, g0):
    full = lambda i: (0, 0)
    return pl.pallas_call(
        _pre_body,
        grid=(N // R,),
        in_specs=[
            pl.BlockSpec((R, D), lambda i: (i, 0)),
            pl.BlockSpec((D, D), full),
            pl.BlockSpec((1, D), full),
            pl.BlockSpec((1, D), full),
            pl.BlockSpec((1, D), full),
            pl.BlockSpec((D, D), full),
        ],
        out_specs=[pl.BlockSpec((R, D), lambda i: (i, 0)),
                   pl.BlockSpec((R, D), lambda i: (i, 0))],
        out_shape=[jax.ShapeDtypeStruct((N, D), jnp.float32),
                   jax.ShapeDtypeStruct((N, D), jnp.float32)],
    )(x, w1, b1, g1, be1, g0)


def _gru_update(p_ref, h_ref, wih_ref, whh_ref, bih_ref, bhh_ref):
    agg = p_ref[...]
    h = h_ref[...]
    gi = lax.dot_general(agg, wih_ref[...], (((1,), (1,)), ((), ())),
                         preferred_element_type=jnp.float32) + bih_ref[...]
    gh = lax.dot_general(h, whh_ref[...], (((1,), (1,)), ((), ())),
                         preferred_element_type=jnp.float32) + bhh_ref[...]
    r = jax.nn.sigmoid(gi[:, :D] + gh[:, :D])
    z = jax.nn.sigmoid(gi[:, D:2 * D] + gh[:, D:2 * D])
    n = jnp.tanh(gi[:, 2 * D:] + r * gh[:, 2 * D:])
    return (1.0 - z) * n + z * h


def _gru_body(p_ref, h_ref, wih_ref, whh_ref, bih_ref, bhh_ref, gn_ref,
              hn_ref, mn_ref):
    hn = _gru_update(p_ref, h_ref, wih_ref, whh_ref, bih_ref, bhh_ref)
    hn_ref[...] = hn
    mn_ref[...] = jnp.dot(hn, gn_ref[...], preferred_element_type=jnp.float32)


def _tc_gru(p, h, wih, whh, bih, bhh, gn):
    full = lambda i: (0, 0)
    return pl.pallas_call(
        _gru_body,
        grid=(N // R,),
        in_specs=[
            pl.BlockSpec((R, D), lambda i: (i, 0)),
            pl.BlockSpec((R, D), lambda i: (i, 0)),
            pl.BlockSpec((3 * D, D), full),
            pl.BlockSpec((3 * D, D), full),
            pl.BlockSpec((1, 3 * D), full),
            pl.BlockSpec((1, 3 * D), full),
            pl.BlockSpec((D, D), full),
        ],
        out_specs=[pl.BlockSpec((R, D), lambda i: (i, 0)),
                   pl.BlockSpec((R, D), lambda i: (i, 0))],
        out_shape=[jax.ShapeDtypeStruct((N, D), jnp.float32),
                   jax.ShapeDtypeStruct((N, D), jnp.float32)],
    )(p, h, wih, whh, bih, bhh, gn)


def _fin_body(h_ref, g2_ref, be2_ref, w2_ref, b2_ref, out_ref, emb_ref):
    hb = h_ref[...] * (g2_ref[...] / jnp.sqrt(1.0 + EPS)) + be2_ref[...]
    emb = lax.dot_general(hb, w2_ref[...], (((1,), (1,)), ((), ())),
                          preferred_element_type=jnp.float32) + b2_ref[...]
    emb_ref[...] = emb
    s = emb - jnp.max(emb, axis=1, keepdims=True)
    out_ref[...] = s - jnp.log(jnp.sum(jnp.exp(s), axis=1, keepdims=True))


def _tc_fin(h, g2, be2, w2, b2):
    full = lambda i: (0, 0)
    return pl.pallas_call(
        _fin_body,
        grid=(N // R,),
        in_specs=[
            pl.BlockSpec((R, D), lambda i: (i, 0)),
            pl.BlockSpec((1, D), full),
            pl.BlockSpec((1, D), full),
            pl.BlockSpec((D, D), full),
            pl.BlockSpec((1, D), full),
        ],
        out_specs=[pl.BlockSpec((R, D), lambda i: (i, 0)),
                   pl.BlockSpec((R, D), lambda i: (i, 0))],
        out_shape=[jax.ShapeDtypeStruct((N, D), jnp.float32),
                   jax.ShapeDtypeStruct((N, D), jnp.float32)],
    )(h, g2, be2, w2, b2)


def kernel(x, edge_index, lin1_W, lin1_b, bn1_gamma, bn1_beta, ggc_weight,
           gru_W_ih, gru_W_hh, gru_b_ih, gru_b_hh, bn2_gamma, bn2_beta,
           lin2_W, lin2_b):
    src = edge_index[0]
    dst = edge_index[1]
    b1 = lin1_b.reshape(1, D)
    g1 = bn1_gamma.reshape(1, D)
    be1 = bn1_beta.reshape(1, D)
    bih = gru_b_ih.reshape(1, 3 * D)
    bhh = gru_b_hh.reshape(1, 3 * D)
    g2 = bn2_gamma.reshape(1, D)
    be2 = bn2_beta.reshape(1, D)
    b2 = lin2_b.reshape(1, D)

    h, m = _tc_pre(x, lin1_W, b1, g1, be1, ggc_weight[0])

    gn_stack = jnp.concatenate([ggc_weight[1:], ggc_weight[:1]], axis=0)

    def step(carry, gn):
        h, m = carry
        p = _sc_segsum(m, src, dst)
        h, m = _tc_gru(p, h, gru_W_ih, gru_W_hh, bih, bhh, gn)
        return (h, m), None

    (h, _), _ = lax.scan(step, (h, m), gn_stack)
    out, emb = _tc_fin(h, g2, be2, lin2_W, b2)
    return (out, emb)
